# Initial kernel scaffold; baseline (speedup 1.0000x reference)
import functools

import jax
import jax.numpy as jnp
from jax import lax
from jax.experimental import pallas as pl
from jax.experimental.pallas import tpu as pltpu
from jax.experimental.pallas import tpu_sc as plsc

_NC = 2
_NS = 16
_NW = _NC * _NS
_CHUNK = 128


def _onehot(b_col, num_seg):
    rows = b_col.shape[0]
    return (lax.broadcasted_iota(jnp.int32, (rows, num_seg), 1) == b_col).astype(
        jnp.float32
    )


def _mm(a, b):
    return jnp.dot(a, b, preferred_element_type=jnp.float32)


def _build_edge_agg(n_pad, emb, cpw):
    rows_per_tile = n_pad // _NS

    mesh = plsc.VectorSubcoreMesh(core_axis_name="c", subcore_axis_name="s")

    @functools.partial(
        pl.kernel,
        out_type=jax.ShapeDtypeStruct((_NC, n_pad, emb), jnp.float32),
        mesh=mesh,
        scratch_types=[
            pltpu.VMEM((cpw, _CHUNK), jnp.int32),
            pltpu.VMEM((cpw, _CHUNK), jnp.int32),
            pltpu.VMEM((_CHUNK, emb), jnp.float32),
            pltpu.VMEM_SHARED((n_pad, emb), jnp.float32),
            pltpu.SemaphoreType.DMA,
        ],
    )
    def edge_agg(r_hbm, src_hbm, dst_hbm, zeros_hbm, out_hbm,
                 src_v, dst_v, gbuf, acc, sem):
        c = lax.axis_index("c")
        s = lax.axis_index("s")
        wid = s * _NC + c
        off = s * rows_per_tile
        pltpu.sync_copy(zeros_hbm.at[pl.ds(off, rows_per_tile)],
                        acc.at[pl.ds(off, rows_per_tile)])
        pltpu.sync_copy(src_hbm.at[pl.ds(wid * cpw, cpw)], src_v)
        pltpu.sync_copy(dst_hbm.at[pl.ds(wid * cpw, cpw)], dst_v)
        plsc.subcore_barrier()

        def body(j, carry):
            pltpu.async_copy(r_hbm.at[src_v.at[j]], gbuf, sem).wait()
            pltpu.sync_copy(gbuf, acc.at[dst_v.at[j]], add=True)
            return carry

        lax.fori_loop(0, cpw, body, 0)
        plsc.subcore_barrier()
        pltpu.sync_copy(acc.at[pl.ds(off, rows_per_tile)],
                        out_hbm.at[c, pl.ds(off, rows_per_tile)])

    return edge_agg


def _pre_call(x_p, b2d, vw1, vb1, vw2, vb2, n_pad, emb, g, rows):
    nb = n_pad // rows
    h2 = vw1.shape[1]

    def body(x_ref, b_ref, w1, b1, w2, b2, r_out, vn_out, pooled):
        i = pl.program_id(0)
        xb = x_ref[...]
        r_out[...] = jnp.maximum(xb, 0.0)
        oh = _onehot(b_ref[...], g)
        contrib = lax.dot_general(oh, xb, (((0,), (0,)), ((), ())),
                                  preferred_element_type=jnp.float32)

        @pl.when(i == 0)
        def _():
            pooled[...] = contrib

        @pl.when(i > 0)
        def _():
            pooled[...] += contrib

        @pl.when(i == nb - 1)
        def _():
            t = jnp.maximum(_mm(pooled[...], w1[...]) + b1[...], 0.0)
            vn_out[...] = jnp.maximum(_mm(t, w2[...]) + b2[...], 0.0)

    return pl.pallas_call(
        body,
        grid=(nb,),
        in_specs=[
            pl.BlockSpec((rows, emb), lambda i: (i, 0)),
            pl.BlockSpec((rows, 1), lambda i: (i, 0)),
            pl.BlockSpec((emb, h2), lambda i: (0, 0)),
            pl.BlockSpec((1, h2), lambda i: (0, 0)),
            pl.BlockSpec((h2, emb), lambda i: (0, 0)),
            pl.BlockSpec((1, emb), lambda i: (0, 0)),
        ],
        out_specs=[
            pl.BlockSpec((rows, emb), lambda i: (i, 0)),
            pl.BlockSpec((g, emb), lambda i: (0, 0)),
        ],
        out_shape=[
            jax.ShapeDtypeStruct((n_pad, emb), jnp.float32),
            jax.ShapeDtypeStruct((g, emb), jnp.float32),
        ],
        scratch_shapes=[pltpu.VMEM((g, emb), jnp.float32)],
    )(x_p, b2d, vw1, vb1, vw2, vb2)


def _qp_call(hin, agg, b2d, eps11, conv, vncur, vnp, n_pad, emb, g, rows):
    nb = n_pad // rows
    h2 = conv["W1"].shape[1]
    has_vn = vnp is not None

    def body(*refs):
        if has_vn:
            (hin_ref, agg0, agg1, b_ref, eps_ref, cw1, cb1, cw2, cb2, scl, bia,
             vnc, vw1, vb1, vw2, vb2, hin_out, r_out, vn_out, pooled) = refs
        else:
            (hin_ref, agg0, agg1, b_ref, eps_ref, cw1, cb1, cw2, cb2, scl, bia,
             vnc, hin_out, r_out, pooled) = refs
        i = pl.program_id(0)
        hb = hin_ref[...]
        z = (1.0 + eps_ref[0, 0]) * hb + agg0[...] + agg1[...]
        t = jnp.maximum(_mm(z, cw1[...]) + cb1[...], 0.0)
        z2 = _mm(t, cw2[...]) + cb2[...]
        z2 = z2 * scl[...] + bia[...]
        h = jnp.maximum(z2, 0.0)
        oh = _onehot(b_ref[...], g)
        hin_new = h + _mm(oh, vnc[...])
        hin_out[...] = hin_new
        r_out[...] = jnp.maximum(hin_new, 0.0)
        contrib = lax.dot_general(oh, hin_new, (((0,), (0,)), ((), ())),
                                  preferred_element_type=jnp.float32)

        @pl.when(i == 0)
        def _():
            pooled[...] = contrib

        @pl.when(i > 0)
        def _():
            pooled[...] += contrib

        if has_vn:
            @pl.when(i == nb - 1)
            def _():
                p = pooled[...] + vnc[...]
                tt = jnp.maximum(_mm(p, vw1[...]) + vb1[...], 0.0)
                vn_out[...] = jnp.maximum(_mm(tt, vw2[...]) + vb2[...], 0.0)

    full = lambda r, c: pl.BlockSpec((r, c), lambda i: (0, 0))
    in_specs = [
        pl.BlockSpec((rows, emb), lambda i: (i, 0)),
        pl.BlockSpec((None, rows, emb), lambda i: (0, i, 0)),
        pl.BlockSpec((None, rows, emb), lambda i: (1, i, 0)),
        pl.BlockSpec((rows, 1), lambda i: (i, 0)),
        pl.BlockSpec(memory_space=pltpu.SMEM),
        full(emb, h2), full(1, h2), full(h2, emb), full(1, emb),
        full(1, emb), full(1, emb),
        full(g, emb),
    ]
    args = [hin, agg, agg, b2d, eps11,
            conv["W1"], conv["b1"].reshape(1, h2),
            conv["W2"], conv["b2"].reshape(1, emb),
            conv["bn_scale"].reshape(1, emb), conv["bn_bias"].reshape(1, emb),
            vncur]
    out_specs = [
        pl.BlockSpec((rows, emb), lambda i: (i, 0)),
        pl.BlockSpec((rows, emb), lambda i: (i, 0)),
    ]
    out_shape = [
        jax.ShapeDtypeStruct((n_pad, emb), jnp.float32),
        jax.ShapeDtypeStruct((n_pad, emb), jnp.float32),
    ]
    if has_vn:
        vh2 = vnp["W1"].shape[1]
        in_specs += [full(emb, vh2), full(1, vh2), full(vh2, emb), full(1, emb)]
        args += [vnp["W1"], vnp["b1"].reshape(1, vh2),
                 vnp["W2"], vnp["b2"].reshape(1, emb)]
        out_specs.append(pl.BlockSpec((g, emb), lambda i: (0, 0)))
        out_shape.append(jax.ShapeDtypeStruct((g, emb), jnp.float32))

    return pl.pallas_call(
        body,
        grid=(nb,),
        in_specs=in_specs,
        out_specs=out_specs,
        out_shape=out_shape,
        scratch_shapes=[pltpu.VMEM((g, emb), jnp.float32)],
    )(*args)


def _q4_call(hin, agg, b2d, eps11, conv, n_pad, emb, g, rows):
    nb = n_pad // rows
    h2 = conv["W1"].shape[1]

    def body(hin_ref, agg0, agg1, b_ref, eps_ref, cw1, cb1, cw2, cb2, scl, bia,
             hg_out, pooled):
        i = pl.program_id(0)
        hb = hin_ref[...]
        z = (1.0 + eps_ref[0, 0]) * hb + agg0[...] + agg1[...]
        t = jnp.maximum(_mm(z, cw1[...]) + cb1[...], 0.0)
        z2 = _mm(t, cw2[...]) + cb2[...]
        z2 = z2 * scl[...] + bia[...]
        oh = _onehot(b_ref[...], g)
        contrib = lax.dot_general(oh, z2, (((0,), (0,)), ((), ())),
                                  preferred_element_type=jnp.float32)

        @pl.when(i == 0)
        def _():
            pooled[...] = contrib

        @pl.when(i > 0)
        def _():
            pooled[...] += contrib

        @pl.when(i == nb - 1)
        def _():
            hg_out[...] = pooled[...]

    full = lambda r, c: pl.BlockSpec((r, c), lambda i: (0, 0))
    return pl.pallas_call(
        body,
        grid=(nb,),
        in_specs=[
            pl.BlockSpec((rows, emb), lambda i: (i, 0)),
            pl.BlockSpec((None, rows, emb), lambda i: (0, i, 0)),
            pl.BlockSpec((None, rows, emb), lambda i: (1, i, 0)),
            pl.BlockSpec((rows, 1), lambda i: (i, 0)),
            pl.BlockSpec(memory_space=pltpu.SMEM),
            full(emb, h2), full(1, h2), full(h2, emb), full(1, emb),
            full(1, emb), full(1, emb),
        ],
        out_specs=pl.BlockSpec((g, emb), lambda i: (0, 0)),
        out_shape=jax.ShapeDtypeStruct((g, emb), jnp.float32),
        scratch_shapes=[pltpu.VMEM((g, emb), jnp.float32)],
    )(hin, agg, agg, b2d, eps11,
      conv["W1"], conv["b1"].reshape(1, h2),
      conv["W2"], conv["b2"].reshape(1, emb),
      conv["bn_scale"].reshape(1, emb), conv["bn_bias"].reshape(1, emb))


def _head_call(hg, fp, pred, emb, g, ncols):
    fpd = fp.shape[1]
    h4 = pred["W1"].shape[1]
    w1a = pred["W1"][:emb]
    w1b = pred["W1"][emb:]
    ntask = pred["W2"].shape[1]
    w2p = jnp.pad(pred["W2"], ((0, 0), (0, ncols - ntask)))
    b2p = jnp.pad(pred["b2"], (0, ncols - ntask)).reshape(1, ncols)

    def body(hg_ref, fp_ref, w1a_ref, w1b_ref, b1_ref, w2_ref, b2_ref, out_ref):
        t = (_mm(hg_ref[...], w1a_ref[...]) + _mm(fp_ref[...], w1b_ref[...])
             + b1_ref[...])
        t = jax.nn.gelu(t)
        out_ref[...] = _mm(t, w2_ref[...]) + b2_ref[...]

    full = lambda r, c: pl.BlockSpec((r, c), lambda i: (0, 0))
    return pl.pallas_call(
        body,
        grid=(1,),
        in_specs=[full(g, emb), full(g, fpd), full(emb, h4), full(fpd, h4),
                  full(1, h4), full(h4, ncols), full(1, ncols)],
        out_specs=full(g, ncols),
        out_shape=jax.ShapeDtypeStruct((g, ncols), jnp.float32),
    )(hg, fp, w1a, w1b, pred["b1"].reshape(1, h4), w2p, b2p)


def kernel(x, edge_index, batch, fp, params):
    n, emb = x.shape
    e = edge_index.shape[1]
    g, fpd = fp.shape
    num_layer = len(params["convs"])

    rows = 2048
    n_pad = -(-(n + 1) // rows) * rows
    cpw = -(-e // (_NW * _CHUNK))
    e_pad = _NW * cpw * _CHUNK

    x_p = jnp.pad(x, ((0, n_pad - n), (0, 0)))
    b2d = jnp.pad(batch, (0, n_pad - n), constant_values=g).reshape(n_pad, 1)
    src_p = jnp.concatenate(
        [edge_index[0], jnp.full((e_pad - e,), n, jnp.int32)]).reshape(-1, _CHUNK)
    dst_p = jnp.concatenate(
        [edge_index[1], jnp.full((e_pad - e,), n, jnp.int32)]).reshape(-1, _CHUNK)
    zeros = jnp.zeros((n_pad, emb), jnp.float32)

    edge_agg = _build_edge_agg(n_pad, emb, cpw)

    vn0 = params["vn"][0]
    r_cur, vncur = _pre_call(
        x_p, b2d, vn0["W1"], vn0["b1"].reshape(1, -1),
        vn0["W2"], vn0["b2"].reshape(1, -1), n_pad, emb, g, rows)
    hin = x_p

    for l in range(num_layer - 1):
        agg = edge_agg(r_cur, src_p, dst_p, zeros)
        conv = params["convs"][l]
        eps11 = (conv["eps"]).reshape(1, 1)
        vnp = params["vn"][l + 1] if l + 1 < num_layer - 1 else None
        outs = _qp_call(hin, agg, b2d, eps11, conv, vncur, vnp,
                        n_pad, emb, g, rows)
        if vnp is not None:
            hin, r_cur, vncur = outs
        else:
            hin, r_cur = outs

    agg = edge_agg(r_cur, src_p, dst_p, zeros)
    conv = params["convs"][num_layer - 1]
    hg = _q4_call(hin, agg, b2d, conv["eps"].reshape(1, 1), conv,
                  n_pad, emb, g, rows)

    ncols = 128
    out_p = _head_call(hg, fp, params["pred"], emb, g, ncols)
    return out_p[:, : params["pred"]["W2"].shape[1]]


# speedup vs baseline: 2.8637x; 2.8637x over previous
import functools

import jax
import jax.numpy as jnp
from jax import lax
from jax.experimental import pallas as pl
from jax.experimental.pallas import tpu as pltpu
from jax.experimental.pallas import tpu_sc as plsc

_NC = 2
_NS = 16
_NW = _NC * _NS
_CHUNK = 128


def _onehot(b_col, num_seg):
    rows = b_col.shape[0]
    return (lax.broadcasted_iota(jnp.int32, (rows, num_seg), 1) == b_col).astype(
        jnp.float32
    )


def _mm(a, b):
    return jnp.dot(a, b, preferred_element_type=jnp.float32)


def _build_edge_agg(n_pad, emb, cpw):
    rows_per_tile = n_pad // _NS

    mesh = plsc.VectorSubcoreMesh(core_axis_name="c", subcore_axis_name="s")

    @functools.partial(
        pl.kernel,
        out_type=jax.ShapeDtypeStruct((_NC, n_pad, emb), jnp.float32),
        mesh=mesh,
        scratch_types=[
            pltpu.VMEM((cpw, _CHUNK), jnp.int32),
            pltpu.VMEM((cpw, _CHUNK), jnp.int32),
            pltpu.VMEM((_CHUNK, emb), jnp.float32),
            pltpu.VMEM_SHARED((n_pad, emb), jnp.float32),
            pltpu.SemaphoreType.DMA,
        ],
    )
    def edge_agg(r_hbm, src_hbm, dst_hbm, zeros_hbm, out_hbm,
                 src_v, dst_v, gbuf, acc, sem):
        c = lax.axis_index("c")
        s = lax.axis_index("s")
        wid = s * _NC + c
        off = s * rows_per_tile
        pltpu.sync_copy(zeros_hbm.at[pl.ds(off, rows_per_tile)],
                        acc.at[pl.ds(off, rows_per_tile)])
        pltpu.sync_copy(src_hbm.at[wid], src_v)
        pltpu.sync_copy(dst_hbm.at[wid], dst_v)
        plsc.subcore_barrier()

        def body(j, carry):
            pltpu.async_copy(r_hbm.at[src_v.at[j]], gbuf, sem).wait()
            pltpu.sync_copy(gbuf, acc.at[dst_v.at[j]], add=True)
            return carry

        lax.fori_loop(0, cpw, body, 0)
        plsc.subcore_barrier()
        pltpu.sync_copy(acc.at[pl.ds(off, rows_per_tile)],
                        out_hbm.at[c, pl.ds(off, rows_per_tile)])

    return edge_agg


def _pre_call(x_p, b2d, vw1, vb1, vw2, vb2, n_pad, emb, g, rows):
    nb = n_pad // rows
    h2 = vw1.shape[1]

    def body(x_ref, b_ref, w1, b1, w2, b2, r_out, vn_out, pooled):
        i = pl.program_id(0)
        xb = x_ref[...]
        r_out[...] = jnp.maximum(xb, 0.0)
        oh = _onehot(b_ref[...], g)
        contrib = lax.dot_general(oh, xb, (((0,), (0,)), ((), ())),
                                  preferred_element_type=jnp.float32)

        @pl.when(i == 0)
        def _():
            pooled[...] = contrib

        @pl.when(i > 0)
        def _():
            pooled[...] += contrib

        @pl.when(i == nb - 1)
        def _():
            t = jnp.maximum(_mm(pooled[...], w1[...]) + b1[...], 0.0)
            vn_out[...] = jnp.maximum(_mm(t, w2[...]) + b2[...], 0.0)

    return pl.pallas_call(
        body,
        grid=(nb,),
        in_specs=[
            pl.BlockSpec((rows, emb), lambda i: (i, 0)),
            pl.BlockSpec((rows, 1), lambda i: (i, 0)),
            pl.BlockSpec((emb, h2), lambda i: (0, 0)),
            pl.BlockSpec((1, h2), lambda i: (0, 0)),
            pl.BlockSpec((h2, emb), lambda i: (0, 0)),
            pl.BlockSpec((1, emb), lambda i: (0, 0)),
        ],
        out_specs=[
            pl.BlockSpec((rows, emb), lambda i: (i, 0)),
            pl.BlockSpec((g, emb), lambda i: (0, 0)),
        ],
        out_shape=[
            jax.ShapeDtypeStruct((n_pad, emb), jnp.float32),
            jax.ShapeDtypeStruct((g, emb), jnp.float32),
        ],
        scratch_shapes=[pltpu.VMEM((g, emb), jnp.float32)],
    )(x_p, b2d, vw1, vb1, vw2, vb2)


def _qp_call(hin, agg, b2d, eps11, conv, vncur, vnp, n_pad, emb, g, rows):
    nb = n_pad // rows
    h2 = conv["W1"].shape[1]
    has_vn = vnp is not None

    def body(*refs):
        if has_vn:
            (hin_ref, agg0, agg1, b_ref, eps_ref, cw1, cb1, cw2, cb2, scl, bia,
             vnc, vw1, vb1, vw2, vb2, hin_out, r_out, vn_out, pooled) = refs
        else:
            (hin_ref, agg0, agg1, b_ref, eps_ref, cw1, cb1, cw2, cb2, scl, bia,
             vnc, hin_out, r_out, pooled) = refs
        i = pl.program_id(0)
        hb = hin_ref[...]
        z = (1.0 + eps_ref[0, 0]) * hb + agg0[...] + agg1[...]
        t = jnp.maximum(_mm(z, cw1[...]) + cb1[...], 0.0)
        z2 = _mm(t, cw2[...]) + cb2[...]
        z2 = z2 * scl[...] + bia[...]
        h = jnp.maximum(z2, 0.0)
        oh = _onehot(b_ref[...], g)
        hin_new = h + _mm(oh, vnc[...])
        hin_out[...] = hin_new
        r_out[...] = jnp.maximum(hin_new, 0.0)
        contrib = lax.dot_general(oh, hin_new, (((0,), (0,)), ((), ())),
                                  preferred_element_type=jnp.float32)

        @pl.when(i == 0)
        def _():
            pooled[...] = contrib

        @pl.when(i > 0)
        def _():
            pooled[...] += contrib

        if has_vn:
            @pl.when(i == nb - 1)
            def _():
                p = pooled[...] + vnc[...]
                tt = jnp.maximum(_mm(p, vw1[...]) + vb1[...], 0.0)
                vn_out[...] = jnp.maximum(_mm(tt, vw2[...]) + vb2[...], 0.0)

    full = lambda r, c: pl.BlockSpec((r, c), lambda i: (0, 0))
    in_specs = [
        pl.BlockSpec((rows, emb), lambda i: (i, 0)),
        pl.BlockSpec((None, rows, emb), lambda i: (0, i, 0)),
        pl.BlockSpec((None, rows, emb), lambda i: (1, i, 0)),
        pl.BlockSpec((rows, 1), lambda i: (i, 0)),
        pl.BlockSpec(memory_space=pltpu.SMEM),
        full(emb, h2), full(1, h2), full(h2, emb), full(1, emb),
        full(1, emb), full(1, emb),
        full(g, emb),
    ]
    args = [hin, agg, agg, b2d, eps11,
            conv["W1"], conv["b1"].reshape(1, h2),
            conv["W2"], conv["b2"].reshape(1, emb),
            conv["bn_scale"].reshape(1, emb), conv["bn_bias"].reshape(1, emb),
            vncur]
    out_specs = [
        pl.BlockSpec((rows, emb), lambda i: (i, 0)),
        pl.BlockSpec((rows, emb), lambda i: (i, 0)),
    ]
    out_shape = [
        jax.ShapeDtypeStruct((n_pad, emb), jnp.float32),
        jax.ShapeDtypeStruct((n_pad, emb), jnp.float32),
    ]
    if has_vn:
        vh2 = vnp["W1"].shape[1]
        in_specs += [full(emb, vh2), full(1, vh2), full(vh2, emb), full(1, emb)]
        args += [vnp["W1"], vnp["b1"].reshape(1, vh2),
                 vnp["W2"], vnp["b2"].reshape(1, emb)]
        out_specs.append(pl.BlockSpec((g, emb), lambda i: (0, 0)))
        out_shape.append(jax.ShapeDtypeStruct((g, emb), jnp.float32))

    return pl.pallas_call(
        body,
        grid=(nb,),
        in_specs=in_specs,
        out_specs=out_specs,
        out_shape=out_shape,
        scratch_shapes=[pltpu.VMEM((g, emb), jnp.float32)],
    )(*args)


def _q4_call(hin, agg, b2d, eps11, conv, n_pad, emb, g, rows):
    nb = n_pad // rows
    h2 = conv["W1"].shape[1]

    def body(hin_ref, agg0, agg1, b_ref, eps_ref, cw1, cb1, cw2, cb2, scl, bia,
             hg_out, pooled):
        i = pl.program_id(0)
        hb = hin_ref[...]
        z = (1.0 + eps_ref[0, 0]) * hb + agg0[...] + agg1[...]
        t = jnp.maximum(_mm(z, cw1[...]) + cb1[...], 0.0)
        z2 = _mm(t, cw2[...]) + cb2[...]
        z2 = z2 * scl[...] + bia[...]
        oh = _onehot(b_ref[...], g)
        contrib = lax.dot_general(oh, z2, (((0,), (0,)), ((), ())),
                                  preferred_element_type=jnp.float32)

        @pl.when(i == 0)
        def _():
            pooled[...] = contrib

        @pl.when(i > 0)
        def _():
            pooled[...] += contrib

        @pl.when(i == nb - 1)
        def _():
            hg_out[...] = pooled[...]

    full = lambda r, c: pl.BlockSpec((r, c), lambda i: (0, 0))
    return pl.pallas_call(
        body,
        grid=(nb,),
        in_specs=[
            pl.BlockSpec((rows, emb), lambda i: (i, 0)),
            pl.BlockSpec((None, rows, emb), lambda i: (0, i, 0)),
            pl.BlockSpec((None, rows, emb), lambda i: (1, i, 0)),
            pl.BlockSpec((rows, 1), lambda i: (i, 0)),
            pl.BlockSpec(memory_space=pltpu.SMEM),
            full(emb, h2), full(1, h2), full(h2, emb), full(1, emb),
            full(1, emb), full(1, emb),
        ],
        out_specs=pl.BlockSpec((g, emb), lambda i: (0, 0)),
        out_shape=jax.ShapeDtypeStruct((g, emb), jnp.float32),
        scratch_shapes=[pltpu.VMEM((g, emb), jnp.float32)],
    )(hin, agg, agg, b2d, eps11,
      conv["W1"], conv["b1"].reshape(1, h2),
      conv["W2"], conv["b2"].reshape(1, emb),
      conv["bn_scale"].reshape(1, emb), conv["bn_bias"].reshape(1, emb))


def _head_call(hg, fp, pred, emb, g, ncols):
    fpd = fp.shape[1]
    h4 = pred["W1"].shape[1]
    w1a = pred["W1"][:emb]
    w1b = pred["W1"][emb:]
    ntask = pred["W2"].shape[1]
    w2p = jnp.pad(pred["W2"], ((0, 0), (0, ncols - ntask)))
    b2p = jnp.pad(pred["b2"], (0, ncols - ntask)).reshape(1, ncols)

    def body(hg_ref, fp_ref, w1a_ref, w1b_ref, b1_ref, w2_ref, b2_ref, out_ref):
        t = (_mm(hg_ref[...], w1a_ref[...]) + _mm(fp_ref[...], w1b_ref[...])
             + b1_ref[...])
        t = jax.nn.gelu(t)
        out_ref[...] = _mm(t, w2_ref[...]) + b2_ref[...]

    full = lambda r, c: pl.BlockSpec((r, c), lambda i: (0, 0))
    return pl.pallas_call(
        body,
        grid=(1,),
        in_specs=[full(g, emb), full(g, fpd), full(emb, h4), full(fpd, h4),
                  full(1, h4), full(h4, ncols), full(1, ncols)],
        out_specs=full(g, ncols),
        out_shape=jax.ShapeDtypeStruct((g, ncols), jnp.float32),
    )(hg, fp, w1a, w1b, pred["b1"].reshape(1, h4), w2p, b2p)


def kernel(x, edge_index, batch, fp, params):
    n, emb = x.shape
    e = edge_index.shape[1]
    g, fpd = fp.shape
    num_layer = len(params["convs"])

    rows = 2048
    n_pad = -(-(n + 1) // rows) * rows
    cpw = -(-(-(-e // (_NW * _CHUNK))) // 8) * 8
    e_pad = _NW * cpw * _CHUNK

    x_p = jnp.pad(x, ((0, n_pad - n), (0, 0)))
    b2d = jnp.pad(batch, (0, n_pad - n), constant_values=g).reshape(n_pad, 1)
    src_p = jnp.concatenate(
        [edge_index[0], jnp.full((e_pad - e,), n, jnp.int32)]
    ).reshape(_NW, cpw, _CHUNK)
    dst_p = jnp.concatenate(
        [edge_index[1], jnp.full((e_pad - e,), n, jnp.int32)]
    ).reshape(_NW, cpw, _CHUNK)
    zeros = jnp.zeros((n_pad, emb), jnp.float32)

    edge_agg = _build_edge_agg(n_pad, emb, cpw)

    vn0 = params["vn"][0]
    r_cur, vncur = _pre_call(
        x_p, b2d, vn0["W1"], vn0["b1"].reshape(1, -1),
        vn0["W2"], vn0["b2"].reshape(1, -1), n_pad, emb, g, rows)
    hin = x_p

    for l in range(num_layer - 1):
        agg = edge_agg(r_cur, src_p, dst_p, zeros)
        conv = params["convs"][l]
        eps11 = (conv["eps"]).reshape(1, 1)
        vnp = params["vn"][l + 1] if l + 1 < num_layer - 1 else None
        outs = _qp_call(hin, agg, b2d, eps11, conv, vncur, vnp,
                        n_pad, emb, g, rows)
        if vnp is not None:
            hin, r_cur, vncur = outs
        else:
            hin, r_cur = outs

    agg = edge_agg(r_cur, src_p, dst_p, zeros)
    conv = params["convs"][num_layer - 1]
    hg = _q4_call(hin, agg, b2d, conv["eps"].reshape(1, 1), conv,
                  n_pad, emb, g, rows)

    ncols = 128
    out_p = _head_call(hg, fp, params["pred"], emb, g, ncols)
    return out_p[:, : params["pred"]["W2"].shape[1]]
